# Initial kernel scaffold; baseline (speedup 1.0000x reference)
import jax
import jax.numpy as jnp
from jax.experimental import pallas as pl


def kernel(x, w_gate, w1, b1, w2, b2):
    raise NotImplementedError("write your pallas kernel here")


# speedup vs baseline: 1.0834x; 1.0834x over previous
import functools

import jax
import jax.numpy as jnp
from jax.experimental import pallas as pl
from jax.experimental.pallas import tpu as pltpu

E = 8
K = 2
D = 1024
DFF = 2048
N = 2048
TOK_TILE = 128
NT = N // TOK_TILE


def _gating_body(x_ref, wg_ref, gates_ref, loss_ref):
    logits = jnp.dot(x_ref[...], wg_ref[...], preferred_element_type=jnp.float32)
    m = jnp.max(logits, axis=1, keepdims=True)
    ex = jnp.exp(logits - m)
    probs = ex / jnp.sum(ex, axis=1, keepdims=True)

    cols = jax.lax.broadcasted_iota(jnp.int32, (N, E), 1)
    max1 = jnp.max(probs, axis=1, keepdims=True)
    idx1 = jnp.min(jnp.where(probs == max1, cols, E), axis=1, keepdims=True)
    m1 = cols == idx1
    p2 = jnp.where(m1, -1.0, probs)
    max2 = jnp.max(p2, axis=1, keepdims=True)
    idx2 = jnp.min(jnp.where(p2 == max2, cols, E), axis=1, keepdims=True)
    m2 = cols == idx2

    denom = max1 + max2
    gates = jnp.where(m1, max1 / denom, 0.0) + jnp.where(m2, max2 / denom, 0.0)
    gates_ref[...] = gates

    importance = jnp.sum(gates, axis=0)
    load = jnp.sum((gates > 0.0).astype(jnp.float32), axis=0)

    def cv_sq(v):
        mean = jnp.sum(v) / E
        var = jnp.sum((v - mean) ** 2) / (E - 1)
        return var / (mean * mean + 1e-10)

    loss_ref[...] = jnp.broadcast_to(
        0.01 * (cv_sq(importance) + cv_sq(load)), (1, 1)
    )


def _expert_body(x_ref, gates_ref, w1_ref, b1_ref, w2_ref, b2_ref, y_ref):
    e = pl.program_id(0)
    t = pl.program_id(1)
    h = jnp.maximum(
        jnp.dot(x_ref[...], w1_ref[0], preferred_element_type=jnp.float32)
        + b1_ref[0],
        0.0,
    )
    o = jnp.dot(h, w2_ref[0], preferred_element_type=jnp.float32) + b2_ref[0]
    cols = jax.lax.broadcasted_iota(jnp.int32, (TOK_TILE, E), 1)
    gcol = jnp.sum(jnp.where(cols == e, gates_ref[...], 0.0), axis=1, keepdims=True)
    rows = pl.ds(t * TOK_TILE, TOK_TILE)

    @pl.when(e == 0)
    def _():
        y_ref[rows, :] = gcol * o

    @pl.when(e != 0)
    def _():
        y_ref[rows, :] = y_ref[rows, :] + gcol * o


def kernel(x, w_gate, w1, b1, w2, b2):
    xf = x.reshape(-1, D)

    gates, loss = pl.pallas_call(
        _gating_body,
        out_shape=(
            jax.ShapeDtypeStruct((N, E), jnp.float32),
            jax.ShapeDtypeStruct((1, 1), jnp.float32),
        ),
        in_specs=[
            pl.BlockSpec((N, D), lambda: (0, 0)),
            pl.BlockSpec((D, E), lambda: (0, 0)),
        ],
        out_specs=(
            pl.BlockSpec((N, E), lambda: (0, 0)),
            pl.BlockSpec((1, 1), lambda: (0, 0)),
        ),
    )(xf, w_gate)

    y = pl.pallas_call(
        _expert_body,
        grid=(E, NT),
        out_shape=jax.ShapeDtypeStruct((N, D), jnp.float32),
        in_specs=[
            pl.BlockSpec((TOK_TILE, D), lambda e, t: (t, 0)),
            pl.BlockSpec((TOK_TILE, E), lambda e, t: (t, 0)),
            pl.BlockSpec((1, D, DFF), lambda e, t: (e, 0, 0)),
            pl.BlockSpec((1, 1, DFF), lambda e, t: (e, 0, 0)),
            pl.BlockSpec((1, DFF, D), lambda e, t: (e, 0, 0)),
            pl.BlockSpec((1, 1, D), lambda e, t: (e, 0, 0)),
        ],
        out_specs=pl.BlockSpec((N, D), lambda e, t: (0, 0)),
    )(xf, gates, w1, b1.reshape(E, 1, DFF), w2, b2.reshape(E, 1, D))

    return (y.reshape(x.shape), loss.reshape(()))
